# baseline (device time: 113098 ns/iter reference)
import jax
import jax.numpy as jnp
from jax import lax
from jax.experimental import pallas as pl
from jax.experimental.pallas import tpu as pltpu

N_DEV = 8
SQ = 1024
SKV_LOC = 1024
H = 8
DH = 128
D = H * DH
CH = SQ // N_DEV
BLK = 64
SCALE = 0.08838834764831843
NEG = -1e9


def kernel(x, Wq, K_ext, V_ext, Wo):
    Qb16 = jnp.dot(x.reshape(SQ, D).astype(jnp.bfloat16),
                   Wq.astype(jnp.bfloat16),
                   preferred_element_type=jnp.float32).astype(jnp.bfloat16)
    K2 = K_ext.reshape(SKV_LOC, D)
    V2 = V_ext.reshape(SKV_LOC, D)

    def body(q_ref, k_ref, v_ref, wo_ref, out_ref,
             o_loc, ml_loc, comm_o, comm_ml,
             send_o, send_ml, send_out, recv_o, recv_ml, recv_out, loc_sem):
        me = lax.axis_index("i")

        bsem = pltpu.get_barrier_semaphore()
        for j in range(N_DEV):
            @pl.when(j != me)
            def _(j=j):
                pl.semaphore_signal(bsem, inc=1, device_id=(j,),
                                    device_id_type=pl.DeviceIdType.MESH)
        pl.semaphore_wait(bsem, N_DEV - 1)

        QC = 4
        QR = SQ // QC
        for qc in range(QC):
            rows = lax.broadcasted_iota(jnp.int32, (QR, SKV_LOC), 0) + qc * QR
            cols = lax.broadcasted_iota(jnp.int32, (QR, SKV_LOC), 1)
            qb = rows // BLK
            kb = me * (SKV_LOC // BLK) + cols // BLK
            mask = (qb == kb) | (kb == 0) | ((qb + kb) % 3 == 0)
            for h in range(H):
                qh = q_ref[qc * QR:(qc + 1) * QR, h * DH:(h + 1) * DH]
                kh = k_ref[:, h * DH:(h + 1) * DH].astype(jnp.bfloat16)
                vh = v_ref[:, h * DH:(h + 1) * DH].astype(jnp.bfloat16)
                s = lax.dot_general(qh, kh, (((1,), (1,)), ((), ())),
                                    preferred_element_type=jnp.float32) * SCALE
                s = jnp.where(mask, s, NEG)
                m = jnp.max(s, axis=1, keepdims=True)
                w = jnp.exp(s - m)
                l = jnp.sum(w, axis=1, keepdims=True)
                o = lax.dot_general(w.astype(jnp.bfloat16), vh,
                                    (((1,), (0,)), ((), ())),
                                    preferred_element_type=jnp.float32)
                o_loc[qc * QR:(qc + 1) * QR, h * DH:(h + 1) * DH] = o
                ml_loc[qc * QR:(qc + 1) * QR, h:h + 1] = m
                ml_loc[qc * QR:(qc + 1) * QR, H + h:H + h + 1] = l

        def o_desc(peer, slot):
            return pltpu.make_async_remote_copy(
                src_ref=o_loc.at[pl.ds(peer * CH, CH), :],
                dst_ref=comm_o.at[slot],
                send_sem=send_o.at[peer],
                recv_sem=recv_o.at[slot],
                device_id=(peer,),
                device_id_type=pl.DeviceIdType.MESH)

        def ml_desc(peer, slot):
            return pltpu.make_async_remote_copy(
                src_ref=ml_loc.at[pl.ds(peer * CH, CH), :],
                dst_ref=comm_ml.at[slot],
                send_sem=send_ml.at[peer],
                recv_sem=recv_ml.at[slot],
                device_id=(peer,),
                device_id_type=pl.DeviceIdType.MESH)

        def out_desc(peer, row, sem):
            return pltpu.make_async_remote_copy(
                src_ref=out_ref.at[pl.ds(row * CH, CH), :],
                dst_ref=out_ref.at[pl.ds(row * CH, CH), :],
                send_sem=send_out.at[peer],
                recv_sem=recv_out.at[sem],
                device_id=(peer,),
                device_id_type=pl.DeviceIdType.MESH)

        for j in range(N_DEV):
            @pl.when(j != me)
            def _(j=j):
                o_desc(j, me).start()
                ml_desc(j, me).start()

        cp_o = pltpu.make_async_copy(
            o_loc.at[pl.ds(me * CH, CH), :], comm_o.at[me], loc_sem)
        cp_o.start()
        cp_o.wait()
        cp_ml = pltpu.make_async_copy(
            ml_loc.at[pl.ds(me * CH, CH), :], comm_ml.at[me], loc_sem)
        cp_ml.start()
        cp_ml.wait()

        for k in range(N_DEV):
            @pl.when(k != me)
            def _(k=k):
                o_desc(k, k).wait_recv()
                ml_desc(k, k).wait_recv()

        ctx_parts = []
        for h in range(H):
            m_acc = comm_ml[0, :, h:h + 1]
            l_acc = comm_ml[0, :, H + h:H + h + 1]
            o_acc = comm_o[0, :, h * DH:(h + 1) * DH]
            for k in range(1, N_DEV):
                mk = comm_ml[k, :, h:h + 1]
                lk = comm_ml[k, :, H + h:H + h + 1]
                ok = comm_o[k, :, h * DH:(h + 1) * DH]
                mn = jnp.maximum(m_acc, mk)
                a = jnp.exp(m_acc - mn)
                b = jnp.exp(mk - mn)
                o_acc = o_acc * a + ok * b
                l_acc = l_acc * a + lk * b
                m_acc = mn
            ctx_parts.append(o_acc / l_acc)
        ctx = jnp.concatenate(ctx_parts, axis=1)

        outc = jnp.dot(ctx.astype(jnp.bfloat16),
                       wo_ref[...].astype(jnp.bfloat16),
                       preferred_element_type=jnp.float32)
        out_ref[pl.ds(me * CH, CH), :] = outc

        for j in range(N_DEV):
            @pl.when(j != me)
            def _(j=j):
                out_desc(j, me, me).start()

        for k in range(N_DEV):
            @pl.when(k != me)
            def _(k=k):
                out_desc(k, k, k).wait_recv()

        for j in range(N_DEV):
            @pl.when(j != me)
            def _(j=j):
                o_desc(j, me).wait_send()
                ml_desc(j, me).wait_send()
                out_desc(j, me, me).wait_send()

    out2 = pl.pallas_call(
        body,
        out_shape=jax.ShapeDtypeStruct((SQ, D), jnp.float32),
        in_specs=[pl.BlockSpec(memory_space=pltpu.VMEM)] * 4,
        out_specs=pl.BlockSpec(memory_space=pltpu.VMEM),
        scratch_shapes=[
            pltpu.VMEM((SQ, D), jnp.float32),
            pltpu.VMEM((SQ, 2 * H), jnp.float32),
            pltpu.VMEM((N_DEV, CH, D), jnp.float32),
            pltpu.VMEM((N_DEV, CH, 2 * H), jnp.float32),
            pltpu.SemaphoreType.DMA((N_DEV,)),
            pltpu.SemaphoreType.DMA((N_DEV,)),
            pltpu.SemaphoreType.DMA((N_DEV,)),
            pltpu.SemaphoreType.DMA((N_DEV,)),
            pltpu.SemaphoreType.DMA((N_DEV,)),
            pltpu.SemaphoreType.DMA((N_DEV,)),
            pltpu.SemaphoreType.DMA,
        ],
        compiler_params=pltpu.CompilerParams(
            collective_id=0, vmem_limit_bytes=60 * 1024 * 1024),
    )(Qb16, K2, V2, Wo)
    return out2.reshape(1, SQ, D)


# device time: 112120 ns/iter; 1.0087x vs baseline; 1.0087x over previous
import jax
import jax.numpy as jnp
from jax import lax
from jax.experimental import pallas as pl
from jax.experimental.pallas import tpu as pltpu

N_DEV = 8
SQ = 1024
SKV_LOC = 1024
H = 8
DH = 128
D = H * DH
CH = SQ // N_DEV
BLK = 64
SCALE = 0.08838834764831843
NEG = -1e9


def kernel(x, Wq, K_ext, V_ext, Wo):
    Qf = jnp.dot(x.reshape(SQ, D), Wq, preferred_element_type=jnp.float32)
    K2 = K_ext.reshape(SKV_LOC, D)
    V2 = V_ext.reshape(SKV_LOC, D)

    def body(q_ref, k_ref, v_ref, wo_ref, out_ref,
             o_loc, ml_loc, comm_o, comm_ml,
             send_o, send_ml, send_out, recv_o, recv_ml, recv_out, loc_sem):
        me = lax.axis_index("i")

        bsem = pltpu.get_barrier_semaphore()
        for j in range(N_DEV):
            @pl.when(j != me)
            def _(j=j):
                pl.semaphore_signal(bsem, inc=1, device_id=(j,),
                                    device_id_type=pl.DeviceIdType.MESH)
        pl.semaphore_wait(bsem, N_DEV - 1)

        QC = 4
        QR = SQ // QC
        for qc in range(QC):
            rows = lax.broadcasted_iota(jnp.int32, (QR, SKV_LOC), 0) + qc * QR
            cols = lax.broadcasted_iota(jnp.int32, (QR, SKV_LOC), 1)
            qb = rows // BLK
            kb = me * (SKV_LOC // BLK) + cols // BLK
            mask = (qb == kb) | (kb == 0) | ((qb + kb) % 3 == 0)
            for h in range(H):
                qh = q_ref[qc * QR:(qc + 1) * QR, h * DH:(h + 1) * DH]
                kh = k_ref[:, h * DH:(h + 1) * DH]
                vh = v_ref[:, h * DH:(h + 1) * DH]
                s = lax.dot_general(qh, kh, (((1,), (1,)), ((), ())),
                                    preferred_element_type=jnp.float32) * SCALE
                s = jnp.where(mask, s, NEG)
                m = jnp.max(s, axis=1, keepdims=True)
                w = jnp.exp(s - m)
                l = jnp.sum(w, axis=1, keepdims=True)
                o = lax.dot_general(w, vh, (((1,), (0,)), ((), ())),
                                    preferred_element_type=jnp.float32)
                o_loc[qc * QR:(qc + 1) * QR, h * DH:(h + 1) * DH] = o
                ml_loc[qc * QR:(qc + 1) * QR, h:h + 1] = m
                ml_loc[qc * QR:(qc + 1) * QR, H + h:H + h + 1] = l

        def o_desc(peer, slot):
            return pltpu.make_async_remote_copy(
                src_ref=o_loc.at[pl.ds(peer * CH, CH), :],
                dst_ref=comm_o.at[slot],
                send_sem=send_o.at[peer],
                recv_sem=recv_o.at[slot],
                device_id=(peer,),
                device_id_type=pl.DeviceIdType.MESH)

        def ml_desc(peer, slot):
            return pltpu.make_async_remote_copy(
                src_ref=ml_loc.at[pl.ds(peer * CH, CH), :],
                dst_ref=comm_ml.at[slot],
                send_sem=send_ml.at[peer],
                recv_sem=recv_ml.at[slot],
                device_id=(peer,),
                device_id_type=pl.DeviceIdType.MESH)

        def out_desc(peer, row, sem):
            return pltpu.make_async_remote_copy(
                src_ref=out_ref.at[pl.ds(row * CH, CH), :],
                dst_ref=out_ref.at[pl.ds(row * CH, CH), :],
                send_sem=send_out.at[peer],
                recv_sem=recv_out.at[sem],
                device_id=(peer,),
                device_id_type=pl.DeviceIdType.MESH)

        for j in range(N_DEV):
            @pl.when(j != me)
            def _(j=j):
                o_desc(j, me).start()
                ml_desc(j, me).start()

        cp_o = pltpu.make_async_copy(
            o_loc.at[pl.ds(me * CH, CH), :], comm_o.at[me], loc_sem)
        cp_o.start()
        cp_o.wait()
        cp_ml = pltpu.make_async_copy(
            ml_loc.at[pl.ds(me * CH, CH), :], comm_ml.at[me], loc_sem)
        cp_ml.start()
        cp_ml.wait()

        for k in range(N_DEV):
            @pl.when(k != me)
            def _(k=k):
                o_desc(k, k).wait_recv()
                ml_desc(k, k).wait_recv()

        ctx_parts = []
        for h in range(H):
            m_acc = comm_ml[0, :, h:h + 1]
            l_acc = comm_ml[0, :, H + h:H + h + 1]
            o_acc = comm_o[0, :, h * DH:(h + 1) * DH]
            for k in range(1, N_DEV):
                mk = comm_ml[k, :, h:h + 1]
                lk = comm_ml[k, :, H + h:H + h + 1]
                ok = comm_o[k, :, h * DH:(h + 1) * DH]
                mn = jnp.maximum(m_acc, mk)
                a = jnp.exp(m_acc - mn)
                b = jnp.exp(mk - mn)
                o_acc = o_acc * a + ok * b
                l_acc = l_acc * a + lk * b
                m_acc = mn
            ctx_parts.append(o_acc / l_acc)
        ctx = jnp.concatenate(ctx_parts, axis=1)

        outc = jnp.dot(ctx, wo_ref[...], preferred_element_type=jnp.float32)
        out_ref[pl.ds(me * CH, CH), :] = outc

        for j in range(N_DEV):
            @pl.when(j != me)
            def _(j=j):
                out_desc(j, me, me).start()

        for k in range(N_DEV):
            @pl.when(k != me)
            def _(k=k):
                out_desc(k, k, k).wait_recv()

        for j in range(N_DEV):
            @pl.when(j != me)
            def _(j=j):
                o_desc(j, me).wait_send()
                ml_desc(j, me).wait_send()
                out_desc(j, me, me).wait_send()

    out2 = pl.pallas_call(
        body,
        out_shape=jax.ShapeDtypeStruct((SQ, D), jnp.float32),
        in_specs=[pl.BlockSpec(memory_space=pltpu.VMEM)] * 4,
        out_specs=pl.BlockSpec(memory_space=pltpu.VMEM),
        scratch_shapes=[
            pltpu.VMEM((SQ, D), jnp.float32),
            pltpu.VMEM((SQ, 2 * H), jnp.float32),
            pltpu.VMEM((N_DEV, CH, D), jnp.float32),
            pltpu.VMEM((N_DEV, CH, 2 * H), jnp.float32),
            pltpu.SemaphoreType.DMA((N_DEV,)),
            pltpu.SemaphoreType.DMA((N_DEV,)),
            pltpu.SemaphoreType.DMA((N_DEV,)),
            pltpu.SemaphoreType.DMA((N_DEV,)),
            pltpu.SemaphoreType.DMA((N_DEV,)),
            pltpu.SemaphoreType.DMA((N_DEV,)),
            pltpu.SemaphoreType.DMA,
        ],
        compiler_params=pltpu.CompilerParams(
            collective_id=0, vmem_limit_bytes=60 * 1024 * 1024),
    )(Qf, K2, V2, Wo)
    return out2.reshape(1, SQ, D)


# device time: 86099 ns/iter; 1.3136x vs baseline; 1.3022x over previous
import jax
import jax.numpy as jnp
from jax import lax
from jax.experimental import pallas as pl
from jax.experimental.pallas import tpu as pltpu

N_DEV = 8
SQ = 1024
SKV_LOC = 1024
H = 8
DH = 128
D = H * DH
CH = SQ // N_DEV
BLK = 64
SCALE = 0.08838834764831843
NEG = -1e9


def kernel(x, Wq, K_ext, V_ext, Wo):
    Qf = jnp.dot(x.reshape(SQ, D), Wq, preferred_element_type=jnp.float32)
    K2 = K_ext.reshape(SKV_LOC, D)
    V2 = V_ext.reshape(SKV_LOC, D)

    def body(q_ref, k_ref, v_ref, wo_ref, out_ref,
             o_loc, ml_loc, comm_o, comm_ml,
             send_o, send_ml, send_out, recv_o, recv_ml, recv_out, loc_sem):
        me = lax.axis_index("i")

        bsem = pltpu.get_barrier_semaphore()
        for j in range(N_DEV):
            @pl.when(j != me)
            def _(j=j):
                pl.semaphore_signal(bsem, inc=1, device_id=(j,),
                                    device_id_type=pl.DeviceIdType.MESH)
        pl.semaphore_wait(bsem, N_DEV - 1)

        def o_desc(peer, slot):
            return pltpu.make_async_remote_copy(
                src_ref=o_loc.at[pl.ds(peer * CH, CH), :],
                dst_ref=comm_o.at[slot],
                send_sem=send_o.at[peer],
                recv_sem=recv_o.at[slot],
                device_id=(peer,),
                device_id_type=pl.DeviceIdType.MESH)

        def ml_desc(peer, slot):
            return pltpu.make_async_remote_copy(
                src_ref=ml_loc.at[pl.ds(peer * CH, CH), :],
                dst_ref=comm_ml.at[slot],
                send_sem=send_ml.at[peer],
                recv_sem=recv_ml.at[slot],
                device_id=(peer,),
                device_id_type=pl.DeviceIdType.MESH)

        def out_desc(peer, row, sem):
            return pltpu.make_async_remote_copy(
                src_ref=out_ref.at[pl.ds(row * CH, CH), :],
                dst_ref=out_ref.at[pl.ds(row * CH, CH), :],
                send_sem=send_out.at[peer],
                recv_sem=recv_out.at[sem],
                device_id=(peer,),
                device_id_type=pl.DeviceIdType.MESH)

        for j in range(N_DEV):
            r0 = j * CH
            qb = (lax.broadcasted_iota(jnp.int32, (CH, 1), 0) + r0) // BLK
            kb = (me * (SKV_LOC // BLK)
                  + lax.broadcasted_iota(jnp.int32, (1, SKV_LOC), 1) // BLK)
            sum3 = qb % 3 + kb % 3
            mask = (qb == kb) | (kb == 0) | (sum3 == 0) | (sum3 == 3)
            for h in range(H):
                qh = q_ref[r0:r0 + CH, h * DH:(h + 1) * DH]
                kh = k_ref[:, h * DH:(h + 1) * DH]
                vh = v_ref[:, h * DH:(h + 1) * DH]
                s = lax.dot_general(qh, kh, (((1,), (1,)), ((), ())),
                                    preferred_element_type=jnp.float32) * SCALE
                s = jnp.where(mask, s, NEG)
                m = jnp.max(s, axis=1, keepdims=True)
                w = jnp.exp(s - m)
                l = jnp.sum(w, axis=1, keepdims=True)
                o = lax.dot_general(w, vh, (((1,), (0,)), ((), ())),
                                    preferred_element_type=jnp.float32)
                o_loc[r0:r0 + CH, h * DH:(h + 1) * DH] = o.astype(jnp.bfloat16)
                ml_loc[r0:r0 + CH, h:h + 1] = m
                ml_loc[r0:r0 + CH, H + h:H + h + 1] = l

            @pl.when(j != me)
            def _(j=j):
                o_desc(j, me).start()
                ml_desc(j, me).start()

        cp_o = pltpu.make_async_copy(
            o_loc.at[pl.ds(me * CH, CH), :], comm_o.at[me], loc_sem)
        cp_o.start()
        cp_o.wait()
        cp_ml = pltpu.make_async_copy(
            ml_loc.at[pl.ds(me * CH, CH), :], comm_ml.at[me], loc_sem)
        cp_ml.start()
        cp_ml.wait()

        for k in range(N_DEV):
            @pl.when(k != me)
            def _(k=k):
                o_desc(k, k).wait_recv()
                ml_desc(k, k).wait_recv()

        ctx_parts = []
        for h in range(H):
            m_acc = comm_ml[0, :, h:h + 1]
            l_acc = comm_ml[0, :, H + h:H + h + 1]
            o_acc = comm_o[0, :, h * DH:(h + 1) * DH]
            for k in range(1, N_DEV):
                mk = comm_ml[k, :, h:h + 1]
                lk = comm_ml[k, :, H + h:H + h + 1]
                ok = comm_o[k, :, h * DH:(h + 1) * DH]
                mn = jnp.maximum(m_acc, mk)
                a = jnp.exp(m_acc - mn)
                b = jnp.exp(mk - mn)
                o_acc = o_acc * a + ok * b
                l_acc = l_acc * a + lk * b
                m_acc = mn
            ctx_parts.append(o_acc / l_acc)
        ctx = jnp.concatenate(ctx_parts, axis=1)

        outc = jnp.dot(ctx, wo_ref[...], preferred_element_type=jnp.float32)
        out_ref[pl.ds(me * CH, CH), :] = outc.astype(jnp.bfloat16)

        for j in range(N_DEV):
            @pl.when(j != me)
            def _(j=j):
                out_desc(j, me, me).start()

        for k in range(N_DEV):
            @pl.when(k != me)
            def _(k=k):
                out_desc(k, k, k).wait_recv()

        for j in range(N_DEV):
            @pl.when(j != me)
            def _(j=j):
                o_desc(j, me).wait_send()
                ml_desc(j, me).wait_send()
                out_desc(j, me, me).wait_send()

    out2 = pl.pallas_call(
        body,
        out_shape=jax.ShapeDtypeStruct((SQ, D), jnp.bfloat16),
        in_specs=[pl.BlockSpec(memory_space=pltpu.VMEM)] * 4,
        out_specs=pl.BlockSpec(memory_space=pltpu.VMEM),
        scratch_shapes=[
            pltpu.VMEM((SQ, D), jnp.bfloat16),
            pltpu.VMEM((SQ, 2 * H), jnp.float32),
            pltpu.VMEM((N_DEV, CH, D), jnp.bfloat16),
            pltpu.VMEM((N_DEV, CH, 2 * H), jnp.float32),
            pltpu.SemaphoreType.DMA((N_DEV,)),
            pltpu.SemaphoreType.DMA((N_DEV,)),
            pltpu.SemaphoreType.DMA((N_DEV,)),
            pltpu.SemaphoreType.DMA((N_DEV,)),
            pltpu.SemaphoreType.DMA((N_DEV,)),
            pltpu.SemaphoreType.DMA((N_DEV,)),
            pltpu.SemaphoreType.DMA,
        ],
        compiler_params=pltpu.CompilerParams(
            collective_id=0, vmem_limit_bytes=60 * 1024 * 1024),
    )(Qf, K2, V2, Wo)
    return out2.astype(jnp.float32).reshape(1, SQ, D)
